# baseline (speedup 1.0000x reference)
import functools

import jax
import jax.numpy as jnp
from jax import lax
from jax.experimental import pallas as pl
from jax.experimental.pallas import tpu as pltpu
from jax.experimental.pallas import tpu_sc as plsc

_NC = 2
_NS = 16
_CH = 128


def _xt_body(x_ref, w_ref, b_ref, o_ref):
    acc = lax.dot_general(x_ref[...], w_ref[0],
                          (((1,), (1,)), ((), ())),
                          preferred_element_type=jnp.float32)
    o_ref[0] = acc + b_ref[0, 0]


def _final_body(x_ref, w_ref, p_ref, b_ref, o_ref):
    acc = lax.dot_general(x_ref[...], w_ref[...],
                          (((1,), (1,)), ((), ())),
                          preferred_element_type=jnp.float32)
    acc = acc + p_ref[0] + p_ref[1] + b_ref[...]
    o_ref[...] = jnp.maximum(acc, 0.0)


def _sc_gather_scatter(xt_flat, gidx, dst, zeros, acc_rows, d):
    nch = gidx.shape[1]
    zr = acc_rows // _NS
    mesh = plsc.VectorSubcoreMesh(core_axis_name="c", subcore_axis_name="s")

    @functools.partial(
        pl.kernel,
        out_type=jax.ShapeDtypeStruct((_NC, acc_rows, d), jnp.float32),
        mesh=mesh,
        scratch_types=[
            pltpu.VMEM((nch, _CH), jnp.int32),
            pltpu.VMEM((2, _CH), jnp.int32),
            pltpu.VMEM((2, _CH, d), jnp.float32),
            pltpu.VMEM_SHARED((acc_rows, d), jnp.float32),
            (pltpu.SemaphoreType.DMA, pltpu.SemaphoreType.DMA),
            (pltpu.SemaphoreType.DMA, pltpu.SemaphoreType.DMA),
            (pltpu.SemaphoreType.DMA, pltpu.SemaphoreType.DMA),
        ],
    )
    def k(xt_hbm, gidx_hbm, dst_hbm, z_hbm, out_hbm,
          gidx_v, dst_v, rows_v, acc_sh, gsems, dsems, ssems):
        c = lax.axis_index("c")
        s = lax.axis_index("s")
        wid = c * _NS + s
        pltpu.sync_copy(z_hbm, acc_sh.at[pl.ds(s * zr, zr)])
        pltpu.sync_copy(gidx_hbm.at[wid], gidx_v)
        plsc.subcore_barrier()

        def gather(j, b):
            pltpu.async_copy(dst_hbm.at[wid, j], dst_v.at[b], dsems[b])
            pltpu.async_copy(xt_hbm.at[gidx_v.at[j]], rows_v.at[b], gsems[b])

        def step(j, b):
            pltpu.make_async_copy(xt_hbm.at[pl.ds(0, _CH)],
                                  rows_v.at[b], gsems[b]).wait()
            pltpu.make_async_copy(dst_hbm.at[wid, 0], dst_v.at[b],
                                  dsems[b]).wait()
            pltpu.async_copy(rows_v.at[b], acc_sh.at[dst_v.at[b]],
                             ssems[b], add=True)
            pltpu.make_async_copy(xt_hbm.at[pl.ds(0, _CH)],
                                  rows_v.at[b], ssems[b]).wait()

            @pl.when(j + 2 < nch)
            def _():
                gather(j + 2, b)

        gather(0, 0)
        gather(1, 1)

        @pl.loop(0, nch // 2)
        def _(g):
            step(g * 2, 0)
            step(g * 2 + 1, 1)

        plsc.subcore_barrier()
        pltpu.sync_copy(acc_sh.at[pl.ds(s * zr, zr)],
                        out_hbm.at[c, pl.ds(s * zr, zr)])

    return k(xt_flat, gidx, dst, zeros)


def kernel(_input, dependency_triples, W_self, b_self, W_dep, b_dep):
    n, d = _input.shape
    two_l = W_dep.shape[0]
    nl = two_l // 2
    e = dependency_triples.shape[0]

    dep = dependency_triples[:, 0]
    lbl = jnp.mod(dependency_triples[:, 1], nl)
    gov = dependency_triples[:, 2]
    gidx = jnp.concatenate([lbl * n + gov, (lbl + nl) * n + dep])
    dst = jnp.concatenate([dep, gov])

    nw = _NC * _NS
    nch = pl.cdiv(2 * e, nw * _CH)
    nch += nch % 2
    per_w = nch * _CH
    pad = per_w * nw - 2 * e
    acc_rows = (n // (8 * _NS) + 1) * (8 * _NS)
    pad_i = jnp.arange(pad, dtype=jnp.int32)
    gidx = jnp.concatenate([gidx, pad_i % (two_l * n)])
    dst = jnp.concatenate([dst, n + pad_i % (acc_rows - n)])
    gidx = gidx.reshape(nw, nch, _CH)
    dst = dst.reshape(nw, nch, _CH)
    zeros = jnp.zeros((acc_rows // _NS, d), jnp.float32)

    bn = 1000
    xt = pl.pallas_call(
        _xt_body,
        grid=(n // bn, two_l),
        in_specs=[
            pl.BlockSpec((bn, d), lambda i, j: (i, 0)),
            pl.BlockSpec((1, d, d), lambda i, j: (j, 0, 0)),
            pl.BlockSpec((1, 1, d), lambda i, j: (j, 0, 0)),
        ],
        out_specs=pl.BlockSpec((1, bn, d), lambda i, j: (j, i, 0)),
        out_shape=jax.ShapeDtypeStruct((two_l, n, d), jnp.float32),
    )(_input.astype(jnp.bfloat16), W_dep.astype(jnp.bfloat16),
      b_dep.reshape(two_l, 1, d))

    parts = _sc_gather_scatter(xt.reshape(two_l * n, d), gidx, dst,
                               zeros, acc_rows, d)

    out = pl.pallas_call(
        _final_body,
        grid=(n // bn,),
        in_specs=[
            pl.BlockSpec((bn, d), lambda i: (i, 0)),
            pl.BlockSpec((d, d), lambda i: (0, 0)),
            pl.BlockSpec((_NC, bn, d), lambda i: (0, i, 0)),
            pl.BlockSpec((1, d), lambda i: (0, 0)),
        ],
        out_specs=pl.BlockSpec((bn, d), lambda i: (i, 0)),
        out_shape=jax.ShapeDtypeStruct((n, d), jnp.float32),
    )(_input, W_self, parts, b_self.reshape(1, d))
    return out


# speedup vs baseline: 1.1947x; 1.1947x over previous
import functools

import jax
import jax.numpy as jnp
from jax import lax
from jax.experimental import pallas as pl
from jax.experimental.pallas import tpu as pltpu
from jax.experimental.pallas import tpu_sc as plsc

_NC = 2
_NS = 16
_CH = 128


def _xt_body(x_ref, w_ref, b_ref, o_ref):
    acc = lax.dot_general(x_ref[...], w_ref[0],
                          (((1,), (1,)), ((), ())),
                          preferred_element_type=jnp.float32)
    o_ref[0] = acc + b_ref[0, 0]


def _final_body(x_ref, w_ref, p_ref, b_ref, o_ref):
    acc = lax.dot_general(x_ref[...], w_ref[...],
                          (((1,), (1,)), ((), ())),
                          preferred_element_type=jnp.float32)
    acc = acc + p_ref[0] + p_ref[1] + b_ref[...]
    o_ref[...] = jnp.maximum(acc, 0.0)


def _sc_gather_scatter(xt_flat, gidx, dst, zeros, acc_rows, d):
    nch = gidx.shape[1]
    zr = acc_rows // _NS
    mesh = plsc.VectorSubcoreMesh(core_axis_name="c", subcore_axis_name="s")

    @functools.partial(
        pl.kernel,
        out_type=jax.ShapeDtypeStruct((_NC, acc_rows, d), jnp.float32),
        mesh=mesh,
        scratch_types=[
            pltpu.VMEM((nch, _CH), jnp.int32),
            pltpu.VMEM((2, _CH), jnp.int32),
            pltpu.VMEM((2, _CH, d), jnp.float32),
            pltpu.VMEM_SHARED((acc_rows, d), jnp.float32),
            (pltpu.SemaphoreType.DMA, pltpu.SemaphoreType.DMA),
            (pltpu.SemaphoreType.DMA, pltpu.SemaphoreType.DMA),
            (pltpu.SemaphoreType.DMA, pltpu.SemaphoreType.DMA),
        ],
    )
    def k(xt_hbm, gidx_hbm, dst_hbm, z_hbm, out_hbm,
          gidx_v, dst_v, rows_v, acc_sh, gsems, dsems, ssems):
        c = lax.axis_index("c")
        s = lax.axis_index("s")
        wid = c * _NS + s
        pltpu.sync_copy(z_hbm, acc_sh.at[pl.ds(s * zr, zr)])
        pltpu.sync_copy(gidx_hbm.at[wid], gidx_v)
        plsc.subcore_barrier()

        def gather(j, b):
            pltpu.async_copy(dst_hbm.at[wid, j], dst_v.at[b], dsems[b])
            pltpu.async_copy(xt_hbm.at[gidx_v.at[j]], rows_v.at[b], gsems[b])

        def step(j, b):
            pltpu.make_async_copy(xt_hbm.at[pl.ds(0, _CH)],
                                  rows_v.at[b], gsems[b]).wait()
            pltpu.make_async_copy(dst_hbm.at[wid, 0], dst_v.at[b],
                                  dsems[b]).wait()
            pltpu.async_copy(rows_v.at[b], acc_sh.at[dst_v.at[b]],
                             ssems[b], add=True)
            pltpu.make_async_copy(xt_hbm.at[pl.ds(0, _CH)],
                                  rows_v.at[b], ssems[b]).wait()

            @pl.when(j + 2 < nch)
            def _():
                gather(j + 2, b)

        gather(0, 0)
        gather(1, 1)

        @pl.loop(0, nch // 2)
        def _(g):
            step(g * 2, 0)
            step(g * 2 + 1, 1)

        plsc.subcore_barrier()
        pltpu.sync_copy(acc_sh.at[pl.ds(s * zr, zr)],
                        out_hbm.at[c, pl.ds(s * zr, zr)])

    return k(xt_flat, gidx, dst, zeros)


def kernel(_input, dependency_triples, W_self, b_self, W_dep, b_dep):
    n, d = _input.shape
    two_l = W_dep.shape[0]
    nl = two_l // 2
    e = dependency_triples.shape[0]

    dep = dependency_triples[:, 0]
    lbl = jnp.mod(dependency_triples[:, 1], nl)
    gov = dependency_triples[:, 2]
    gidx = jnp.concatenate([lbl * n + gov, (lbl + nl) * n + dep])
    dst = jnp.concatenate([dep, gov])

    nw = _NC * _NS
    nch = pl.cdiv(2 * e, nw * _CH)
    nch += nch % 2
    per_w = nch * _CH
    pad = per_w * nw - 2 * e
    acc_rows = (n // (8 * _NS) + 1) * (8 * _NS)
    pad_i = jnp.arange(pad, dtype=jnp.int32)
    gidx = jnp.concatenate([gidx, pad_i % (two_l * n)])
    dst = jnp.concatenate([dst, n + pad_i % (acc_rows - n)])
    gidx = gidx.reshape(nw, nch, _CH)
    dst = dst.reshape(nw, nch, _CH)
    zeros = jnp.zeros((acc_rows // _NS, d), jnp.float32)

    bn = 1000
    bnx = 2000
    xt = pl.pallas_call(
        _xt_body,
        grid=(n // bnx, two_l),
        in_specs=[
            pl.BlockSpec((bnx, d), lambda i, j: (i, 0)),
            pl.BlockSpec((1, d, d), lambda i, j: (j, 0, 0)),
            pl.BlockSpec((1, 1, d), lambda i, j: (j, 0, 0)),
        ],
        out_specs=pl.BlockSpec((1, bnx, d), lambda i, j: (j, i, 0)),
        out_shape=jax.ShapeDtypeStruct((two_l, n, d), jnp.float32),
    )(_input.astype(jnp.bfloat16), W_dep.astype(jnp.bfloat16),
      b_dep.reshape(two_l, 1, d))

    parts = _sc_gather_scatter(xt.reshape(two_l * n, d), gidx, dst,
                               zeros, acc_rows, d)

    out = pl.pallas_call(
        _final_body,
        grid=(n // bn,),
        in_specs=[
            pl.BlockSpec((bn, d), lambda i: (i, 0)),
            pl.BlockSpec((d, d), lambda i: (0, 0)),
            pl.BlockSpec((_NC, bn, d), lambda i: (0, i, 0)),
            pl.BlockSpec((1, d), lambda i: (0, 0)),
        ],
        out_specs=pl.BlockSpec((bn, d), lambda i: (i, 0)),
        out_shape=jax.ShapeDtypeStruct((n, d), jnp.float32),
    )(_input, W_self, parts, b_self.reshape(1, d))
    return out


# speedup vs baseline: 1.3585x; 1.1371x over previous
import functools

import jax
import jax.numpy as jnp
from jax import lax
from jax.experimental import pallas as pl
from jax.experimental.pallas import tpu as pltpu
from jax.experimental.pallas import tpu_sc as plsc

_NC = 2
_NS = 16
_CH = 128


def _xt_body(x_ref, w_ref, b_ref, o_ref):
    acc = lax.dot_general(x_ref[...], w_ref[0],
                          (((1,), (1,)), ((), ())),
                          preferred_element_type=jnp.float32)
    o_ref[0] = acc + b_ref[0, 0]


def _final_body(x_ref, w_ref, p_ref, b_ref, o_ref):
    acc = lax.dot_general(x_ref[...], w_ref[...],
                          (((1,), (1,)), ((), ())),
                          preferred_element_type=jnp.float32)
    acc = acc + p_ref[0] + p_ref[1] + b_ref[...]
    o_ref[...] = jnp.maximum(acc, 0.0)


def _sc_gather_scatter(xt_flat, gidx, dst, zeros, acc_rows, d):
    nch = gidx.shape[1]
    zr = acc_rows // _NS
    mesh = plsc.VectorSubcoreMesh(core_axis_name="c", subcore_axis_name="s")

    @functools.partial(
        pl.kernel,
        out_type=jax.ShapeDtypeStruct((_NC, acc_rows, d), jnp.float32),
        mesh=mesh,
        scratch_types=[
            pltpu.VMEM((nch, _CH), jnp.int32),
            pltpu.VMEM((2, _CH), jnp.int32),
            pltpu.VMEM((2, _CH, d), jnp.float32),
            pltpu.VMEM_SHARED((acc_rows, d), jnp.float32),
            (pltpu.SemaphoreType.DMA, pltpu.SemaphoreType.DMA),
            (pltpu.SemaphoreType.DMA, pltpu.SemaphoreType.DMA),
            (pltpu.SemaphoreType.DMA, pltpu.SemaphoreType.DMA),
        ],
    )
    def k(xt_hbm, gidx_hbm, dst_hbm, z_hbm, out_hbm,
          gidx_v, dst_v, rows_v, acc_sh, gsems, dsems, ssems):
        c = lax.axis_index("c")
        s = lax.axis_index("s")
        wid = c * _NS + s
        pltpu.sync_copy(z_hbm, acc_sh.at[pl.ds(s * zr, zr)])
        pltpu.sync_copy(gidx_hbm.at[wid], gidx_v)
        plsc.subcore_barrier()

        def gather(j, b):
            pltpu.async_copy(dst_hbm.at[wid, j], dst_v.at[b], dsems[b])
            pltpu.async_copy(xt_hbm.at[gidx_v.at[j]], rows_v.at[b], gsems[b])

        def step(j, b):
            pltpu.make_async_copy(xt_hbm.at[pl.ds(0, _CH)],
                                  rows_v.at[b], gsems[b]).wait()
            pltpu.make_async_copy(dst_hbm.at[wid, 0], dst_v.at[b],
                                  dsems[b]).wait()
            pltpu.async_copy(rows_v.at[b], acc_sh.at[dst_v.at[b]],
                             ssems[b], add=True)
            pltpu.make_async_copy(xt_hbm.at[pl.ds(0, _CH)],
                                  rows_v.at[b], ssems[b]).wait()

            @pl.when(j + 2 < nch)
            def _():
                gather(j + 2, b)

        gather(0, 0)
        gather(1, 1)

        @pl.loop(0, nch // 2)
        def _(g):
            step(g * 2, 0)
            step(g * 2 + 1, 1)

        plsc.subcore_barrier()
        pltpu.sync_copy(acc_sh.at[pl.ds(s * zr, zr)],
                        out_hbm.at[c, pl.ds(s * zr, zr)])

    return k(xt_flat, gidx, dst, zeros)


def kernel(_input, dependency_triples, W_self, b_self, W_dep, b_dep):
    n, d = _input.shape
    two_l = W_dep.shape[0]
    nl = two_l // 2
    e = dependency_triples.shape[0]

    dep = dependency_triples[:, 0]
    lbl = jnp.mod(dependency_triples[:, 1], nl)
    gov = dependency_triples[:, 2]
    gidx = jnp.concatenate([lbl * n + gov, (lbl + nl) * n + dep])
    dst = jnp.concatenate([dep, gov])

    nw = _NC * _NS
    nch = pl.cdiv(2 * e, nw * _CH)
    nch += nch % 2
    per_w = nch * _CH
    pad = per_w * nw - 2 * e
    acc_rows = (n // (8 * _NS) + 1) * (8 * _NS)
    pad_i = jnp.arange(pad, dtype=jnp.int32)
    gidx = jnp.concatenate([gidx, pad_i % (two_l * n)])
    dst = jnp.concatenate([dst, n + pad_i % (acc_rows - n)])
    gidx = gidx.reshape(nw, nch, _CH)
    dst = dst.reshape(nw, nch, _CH)
    zeros = jnp.zeros((acc_rows // _NS, d), jnp.float32)

    bn = 1000
    bnx = 5000
    xt = pl.pallas_call(
        _xt_body,
        grid=(n // bnx, two_l),
        in_specs=[
            pl.BlockSpec((bnx, d), lambda i, j: (i, 0)),
            pl.BlockSpec((1, d, d), lambda i, j: (j, 0, 0)),
            pl.BlockSpec((1, 1, d), lambda i, j: (j, 0, 0)),
        ],
        out_specs=pl.BlockSpec((1, bnx, d), lambda i, j: (j, i, 0)),
        out_shape=jax.ShapeDtypeStruct((two_l, n, d), jnp.float32),
    )(_input.astype(jnp.bfloat16), W_dep.astype(jnp.bfloat16),
      b_dep.reshape(two_l, 1, d))

    parts = _sc_gather_scatter(xt.reshape(two_l * n, d), gidx, dst,
                               zeros, acc_rows, d)

    out = pl.pallas_call(
        _final_body,
        grid=(n // bn,),
        in_specs=[
            pl.BlockSpec((bn, d), lambda i: (i, 0)),
            pl.BlockSpec((d, d), lambda i: (0, 0)),
            pl.BlockSpec((_NC, bn, d), lambda i: (0, i, 0)),
            pl.BlockSpec((1, d), lambda i: (0, 0)),
        ],
        out_specs=pl.BlockSpec((bn, d), lambda i: (i, 0)),
        out_shape=jax.ShapeDtypeStruct((n, d), jnp.float32),
    )(_input, W_self, parts, b_self.reshape(1, d))
    return out


# speedup vs baseline: 1.4121x; 1.0394x over previous
import functools

import jax
import jax.numpy as jnp
from jax import lax
from jax.experimental import pallas as pl
from jax.experimental.pallas import tpu as pltpu
from jax.experimental.pallas import tpu_sc as plsc

_NC = 2
_NS = 16
_CH = 128


def _xt_body(x_ref, w_ref, b_ref, o_ref):
    acc = lax.dot_general(x_ref[...], w_ref[0],
                          (((1,), (1,)), ((), ())),
                          preferred_element_type=jnp.float32)
    o_ref[0] = acc + b_ref[0, 0]


def _final_body(x_ref, w_ref, p_ref, b_ref, o_ref):
    acc = lax.dot_general(x_ref[...], w_ref[...],
                          (((1,), (1,)), ((), ())),
                          preferred_element_type=jnp.float32)
    acc = acc + p_ref[0] + p_ref[1] + b_ref[...]
    o_ref[...] = jnp.maximum(acc, 0.0)


def _sc_gather_scatter(xt_flat, gidx, dst, zeros, acc_rows, d):
    nch = gidx.shape[1]
    zr = acc_rows // _NS
    mesh = plsc.VectorSubcoreMesh(core_axis_name="c", subcore_axis_name="s")

    @functools.partial(
        pl.kernel,
        out_type=jax.ShapeDtypeStruct((_NC, acc_rows, d), jnp.float32),
        mesh=mesh,
        scratch_types=[
            pltpu.VMEM((nch, _CH), jnp.int32),
            pltpu.VMEM((2, _CH), jnp.int32),
            pltpu.VMEM((2, _CH, d), jnp.float32),
            pltpu.VMEM_SHARED((acc_rows, d), jnp.float32),
            (pltpu.SemaphoreType.DMA, pltpu.SemaphoreType.DMA),
            (pltpu.SemaphoreType.DMA, pltpu.SemaphoreType.DMA),
            (pltpu.SemaphoreType.DMA, pltpu.SemaphoreType.DMA),
        ],
    )
    def k(xt_hbm, gidx_hbm, dst_hbm, z_hbm, out_hbm,
          gidx_v, dst_v, rows_v, acc_sh, gsems, dsems, ssems):
        c = lax.axis_index("c")
        s = lax.axis_index("s")
        wid = c * _NS + s
        pltpu.sync_copy(z_hbm, acc_sh.at[pl.ds(s * zr, zr)])
        pltpu.sync_copy(gidx_hbm.at[wid], gidx_v)
        plsc.subcore_barrier()

        def gather(j, b):
            pltpu.async_copy(dst_hbm.at[wid, j], dst_v.at[b], dsems[b])
            pltpu.async_copy(xt_hbm.at[gidx_v.at[j]], rows_v.at[b], gsems[b])

        def step(j, b):
            pltpu.make_async_copy(xt_hbm.at[pl.ds(0, _CH)],
                                  rows_v.at[b], gsems[b]).wait()
            pltpu.make_async_copy(dst_hbm.at[wid, 0], dst_v.at[b],
                                  dsems[b]).wait()
            pltpu.async_copy(rows_v.at[b], acc_sh.at[dst_v.at[b]],
                             ssems[b], add=True)
            pltpu.make_async_copy(xt_hbm.at[pl.ds(0, _CH)],
                                  rows_v.at[b], ssems[b]).wait()

            @pl.when(j + 2 < nch)
            def _():
                gather(j + 2, b)

        gather(0, 0)
        gather(1, 1)

        @pl.loop(0, nch // 2)
        def _(g):
            step(g * 2, 0)
            step(g * 2 + 1, 1)

        plsc.subcore_barrier()
        pltpu.sync_copy(acc_sh.at[pl.ds(s * zr, zr)],
                        out_hbm.at[c, pl.ds(s * zr, zr)])

    return k(xt_flat, gidx, dst, zeros)


def kernel(_input, dependency_triples, W_self, b_self, W_dep, b_dep):
    n, d = _input.shape
    two_l = W_dep.shape[0]
    nl = two_l // 2
    e = dependency_triples.shape[0]

    dep = dependency_triples[:, 0]
    lbl = jnp.mod(dependency_triples[:, 1], nl)
    gov = dependency_triples[:, 2]
    gidx = jnp.concatenate([lbl * n + gov, (lbl + nl) * n + dep])
    dst = jnp.concatenate([dep, gov])

    nw = _NC * _NS
    nch = pl.cdiv(2 * e, nw * _CH)
    nch += nch % 2
    per_w = nch * _CH
    pad = per_w * nw - 2 * e
    acc_rows = (n // (8 * _NS) + 1) * (8 * _NS)
    pad_i = jnp.arange(pad, dtype=jnp.int32)
    gidx = jnp.concatenate([gidx, pad_i % (two_l * n)])
    dst = jnp.concatenate([dst, n + pad_i % (acc_rows - n)])
    gidx = gidx.reshape(nw, nch, _CH)
    dst = dst.reshape(nw, nch, _CH)
    zeros = jnp.zeros((acc_rows // _NS, d), jnp.float32)

    bn = 1000
    bnx = 10000
    xt = pl.pallas_call(
        _xt_body,
        grid=(n // bnx, two_l),
        in_specs=[
            pl.BlockSpec((bnx, d), lambda i, j: (i, 0)),
            pl.BlockSpec((1, d, d), lambda i, j: (j, 0, 0)),
            pl.BlockSpec((1, 1, d), lambda i, j: (j, 0, 0)),
        ],
        out_specs=pl.BlockSpec((1, bnx, d), lambda i, j: (j, i, 0)),
        out_shape=jax.ShapeDtypeStruct((two_l, n, d), jnp.float32),
    )(_input.astype(jnp.bfloat16), W_dep.astype(jnp.bfloat16),
      b_dep.reshape(two_l, 1, d))

    parts = _sc_gather_scatter(xt.reshape(two_l * n, d), gidx, dst,
                               zeros, acc_rows, d)

    out = pl.pallas_call(
        _final_body,
        grid=(n // bn,),
        in_specs=[
            pl.BlockSpec((bn, d), lambda i: (i, 0)),
            pl.BlockSpec((d, d), lambda i: (0, 0)),
            pl.BlockSpec((_NC, bn, d), lambda i: (0, i, 0)),
            pl.BlockSpec((1, d), lambda i: (0, 0)),
        ],
        out_specs=pl.BlockSpec((bn, d), lambda i: (i, 0)),
        out_shape=jax.ShapeDtypeStruct((n, d), jnp.float32),
    )(_input, W_self, parts, b_self.reshape(1, d))
    return out


# speedup vs baseline: 1.4322x; 1.0142x over previous
import functools

import jax
import jax.numpy as jnp
from jax import lax
from jax.experimental import pallas as pl
from jax.experimental.pallas import tpu as pltpu
from jax.experimental.pallas import tpu_sc as plsc

_NC = 2
_NS = 16
_CH = 128


def _xt_body(x_ref, w_ref, b_ref, o_ref):
    acc = lax.dot_general(x_ref[...], w_ref[0],
                          (((1,), (1,)), ((), ())),
                          preferred_element_type=jnp.float32)
    o_ref[0] = acc + b_ref[0, 0]


def _final_body(x_ref, w_ref, p_ref, b_ref, o_ref):
    acc = lax.dot_general(x_ref[...], w_ref[...],
                          (((1,), (1,)), ((), ())),
                          preferred_element_type=jnp.float32)
    acc = acc + p_ref[0] + p_ref[1] + b_ref[...]
    o_ref[...] = jnp.maximum(acc, 0.0)


def _sc_gather_scatter(xt_flat, gidx, dst, zeros, acc_rows, d):
    nch = gidx.shape[1]
    zr = acc_rows // _NS
    mesh = plsc.VectorSubcoreMesh(core_axis_name="c", subcore_axis_name="s")

    @functools.partial(
        pl.kernel,
        out_type=jax.ShapeDtypeStruct((_NC, acc_rows, d), jnp.float32),
        mesh=mesh,
        scratch_types=[
            pltpu.VMEM((nch, _CH), jnp.int32),
            pltpu.VMEM((2, _CH), jnp.int32),
            pltpu.VMEM((2, _CH, d), jnp.float32),
            pltpu.VMEM_SHARED((acc_rows, d), jnp.float32),
            (pltpu.SemaphoreType.DMA, pltpu.SemaphoreType.DMA),
            (pltpu.SemaphoreType.DMA, pltpu.SemaphoreType.DMA),
            (pltpu.SemaphoreType.DMA, pltpu.SemaphoreType.DMA),
        ],
    )
    def k(xt_hbm, gidx_hbm, dst_hbm, z_hbm, out_hbm,
          gidx_v, dst_v, rows_v, acc_sh, gsems, dsems, ssems):
        c = lax.axis_index("c")
        s = lax.axis_index("s")
        wid = c * _NS + s
        pltpu.sync_copy(z_hbm, acc_sh.at[pl.ds(s * zr, zr)])
        pltpu.sync_copy(gidx_hbm.at[wid], gidx_v)
        plsc.subcore_barrier()

        def gather(j, b):
            pltpu.async_copy(dst_hbm.at[wid, j], dst_v.at[b], dsems[b])
            pltpu.async_copy(xt_hbm.at[gidx_v.at[j]], rows_v.at[b], gsems[b])

        def step(j, b):
            pltpu.make_async_copy(xt_hbm.at[pl.ds(0, _CH)],
                                  rows_v.at[b], gsems[b]).wait()
            pltpu.make_async_copy(dst_hbm.at[wid, 0], dst_v.at[b],
                                  dsems[b]).wait()
            pltpu.async_copy(rows_v.at[b], acc_sh.at[dst_v.at[b]],
                             ssems[b], add=True)
            pltpu.make_async_copy(xt_hbm.at[pl.ds(0, _CH)],
                                  rows_v.at[b], ssems[b]).wait()

            @pl.when(j + 2 < nch)
            def _():
                gather(j + 2, b)

        gather(0, 0)
        gather(1, 1)

        @pl.loop(0, nch // 2)
        def _(g):
            step(g * 2, 0)
            step(g * 2 + 1, 1)

        plsc.subcore_barrier()
        pltpu.sync_copy(acc_sh.at[pl.ds(s * zr, zr)],
                        out_hbm.at[c, pl.ds(s * zr, zr)])

    return k(xt_flat, gidx, dst, zeros)


def kernel(_input, dependency_triples, W_self, b_self, W_dep, b_dep):
    n, d = _input.shape
    two_l = W_dep.shape[0]
    nl = two_l // 2
    e = dependency_triples.shape[0]

    dep = dependency_triples[:, 0]
    lbl = jnp.mod(dependency_triples[:, 1], nl)
    gov = dependency_triples[:, 2]
    gidx = jnp.concatenate([lbl * n + gov, (lbl + nl) * n + dep])
    dst = jnp.concatenate([dep, gov])

    nw = _NC * _NS
    nch = pl.cdiv(2 * e, nw * _CH)
    nch += nch % 2
    per_w = nch * _CH
    pad = per_w * nw - 2 * e
    acc_rows = (n // (8 * _NS) + 1) * (8 * _NS)
    pad_i = jnp.arange(pad, dtype=jnp.int32)
    gidx = jnp.concatenate([gidx, pad_i % (two_l * n)])
    dst = jnp.concatenate([dst, n + pad_i % (acc_rows - n)])
    gidx = gidx.reshape(nw, nch, _CH)
    dst = dst.reshape(nw, nch, _CH)
    zeros = jnp.zeros((acc_rows // _NS, d), jnp.float32)

    bn = 2000
    bnx = 10000
    xt = pl.pallas_call(
        _xt_body,
        grid=(n // bnx, two_l),
        in_specs=[
            pl.BlockSpec((bnx, d), lambda i, j: (i, 0)),
            pl.BlockSpec((1, d, d), lambda i, j: (j, 0, 0)),
            pl.BlockSpec((1, 1, d), lambda i, j: (j, 0, 0)),
        ],
        out_specs=pl.BlockSpec((1, bnx, d), lambda i, j: (j, i, 0)),
        out_shape=jax.ShapeDtypeStruct((two_l, n, d), jnp.float32),
    )(_input.astype(jnp.bfloat16), W_dep.astype(jnp.bfloat16),
      b_dep.reshape(two_l, 1, d))

    parts = _sc_gather_scatter(xt.reshape(two_l * n, d), gidx, dst,
                               zeros, acc_rows, d)

    out = pl.pallas_call(
        _final_body,
        grid=(n // bn,),
        in_specs=[
            pl.BlockSpec((bn, d), lambda i: (i, 0)),
            pl.BlockSpec((d, d), lambda i: (0, 0)),
            pl.BlockSpec((_NC, bn, d), lambda i: (0, i, 0)),
            pl.BlockSpec((1, d), lambda i: (0, 0)),
        ],
        out_specs=pl.BlockSpec((bn, d), lambda i: (i, 0)),
        out_shape=jax.ShapeDtypeStruct((n, d), jnp.float32),
    )(_input, W_self, parts, b_self.reshape(1, d))
    return out
